# baseline (speedup 1.0000x reference)
import functools

import jax
import jax.numpy as jnp
from jax import lax
from jax.experimental import pallas as pl
from jax.experimental.pallas import tpu as pltpu
from jax.experimental.pallas import tpu_sc as plsc

B = 16384
F = 26
V = 100000
D = 16

NC = 2
NS = 16
NW = NC * NS

PAIRS = F * D
PAIRS_W = PAIRS // NW
VLOW = 74880
VHI = V - VLOW
QB = 4096
NQ = B // QB
UNROLL = 8


def _emb_kernel(xt_hbm, tt_hbm, ot_hbm, abuf, bbuf, idxbuf, val0, val1,
                sem_a, sem_b, sem_o):
    wid = lax.axis_index("s") * NC + lax.axis_index("c")
    p0 = wid * PAIRS_W
    vals = (val0, val1)

    def wait_a():
        pltpu.make_async_copy(
            tt_hbm.at[0, 0, pl.ds(0, VLOW)], abuf, sem_a).wait()

    def wait_b():
        pltpu.make_async_copy(
            tt_hbm.at[0, 0, pl.ds(VLOW, VHI)], bbuf, sem_b).wait()

    def wait_out(slot):
        pltpu.make_async_copy(
            vals[slot], ot_hbm.at[0, 0, pl.ds(0, QB)], sem_o.at[slot]).wait()

    def pass_a(q, slot):
        vs = vals[slot]

        def body(jj, _):
            o = jj * (16 * UNROLL)
            for u in range(UNROLL):
                oo = o + u * 16
                v = idxbuf[pl.ds(q * QB + oo, 16)]
                vc = jnp.minimum(v, VLOW - 1)
                vs[pl.ds(oo, 16)] = plsc.load_gather(abuf, [vc])
            return 0

        lax.fori_loop(0, QB // (16 * UNROLL), body, 0)

    def pass_b(q, slot, f, d):
        vs = vals[slot]

        def body(jj, _):
            o = jj * (16 * UNROLL)
            for u in range(UNROLL):
                oo = o + u * 16
                v = idxbuf[pl.ds(q * QB + oo, 16)]
                hi = v >= VLOW
                vc = jnp.maximum(v - VLOW, 0)
                g = plsc.load_gather(bbuf, [vc])
                vs[pl.ds(oo, 16)] = jnp.where(hi, g, vs[pl.ds(oo, 16)])
            return 0

        lax.fori_loop(0, QB // (16 * UNROLL), body, 0)
        pltpu.async_copy(vs, ot_hbm.at[f, d, pl.ds(q * QB, QB)],
                         sem_o.at[slot])

    f0 = lax.div(p0, D)
    d0 = lax.rem(p0, D)
    pltpu.sync_copy(xt_hbm.at[f0, pl.ds(0, B)], idxbuf)
    pltpu.async_copy(tt_hbm.at[f0, d0, pl.ds(0, VLOW)], abuf, sem_a)
    pltpu.async_copy(tt_hbm.at[f0, d0, pl.ds(VLOW, VHI)], bbuf, sem_b)

    def plane(k, _):
        p = p0 + k
        f = lax.div(p, D)
        d = lax.rem(p, D)
        fprev = lax.div(p - 1, D)
        fn = lax.div(p + 1, D)
        dn = lax.rem(p + 1, D)
        not_last = k < PAIRS_W - 1
        fn = jnp.minimum(fn, F - 1)

        @pl.when((k > 0) & (f != fprev))
        def _():
            pltpu.sync_copy(xt_hbm.at[f, pl.ds(0, B)], idxbuf)

        wait_a()
        @pl.when(k > 0)
        def _():
            wait_out(0)
        pass_a(0, 0)

        @pl.when(k > 0)
        def _():
            wait_out(1)
        pass_a(1, 1)
        wait_b()
        pass_b(0, 0, f, d)
        pass_b(1, 1, f, d)
        wait_out(0)
        pass_a(2, 0)
        wait_out(1)
        pass_a(3, 1)

        @pl.when(not_last)
        def _():
            pltpu.async_copy(tt_hbm.at[fn, dn, pl.ds(0, VLOW)], abuf, sem_a)

        pass_b(2, 0, f, d)
        pass_b(3, 1, f, d)

        @pl.when(not_last)
        def _():
            pltpu.async_copy(tt_hbm.at[fn, dn, pl.ds(VLOW, VHI)], bbuf, sem_b)

        return 0

    lax.fori_loop(0, PAIRS_W, plane, 0)
    wait_out(0)
    wait_out(1)


@jax.jit
def kernel(X, tables):
    xt = X.T
    tt = jnp.transpose(tables, (0, 2, 1))
    mesh = plsc.VectorSubcoreMesh(core_axis_name="c", subcore_axis_name="s")
    ot = pl.kernel(
        _emb_kernel,
        out_type=jax.ShapeDtypeStruct((F, D, B), jnp.float32),
        mesh=mesh,
        scratch_types=[
            pltpu.VMEM((VLOW,), jnp.float32),
            pltpu.VMEM((VHI,), jnp.float32),
            pltpu.VMEM((B,), jnp.int32),
            pltpu.VMEM((QB,), jnp.float32),
            pltpu.VMEM((QB,), jnp.float32),
            pltpu.SemaphoreType.DMA,
            pltpu.SemaphoreType.DMA,
            pltpu.SemaphoreType.DMA((2,)),
        ],
        compiler_params=pltpu.CompilerParams(needs_layout_passes=False),
    )(xt, tt)
    return jnp.transpose(ot, (2, 0, 1))


# speedup vs baseline: 1.1457x; 1.1457x over previous
import functools

import jax
import jax.numpy as jnp
from jax import lax
from jax.experimental import pallas as pl
from jax.experimental.pallas import tpu as pltpu
from jax.experimental.pallas import tpu_sc as plsc

B = 16384
F = 26
V = 100000
D = 16

NC = 2
NS = 16
NW = NC * NS

PAIRS = F * D
PAIRS_W = PAIRS // NW
IB = 4096
NCH = B // IB
UNROLL = 16


def _emb_kernel(xt_hbm, tt_hbm, ot_hbm, rowbuf, idxbuf, valbuf0, valbuf1, sem_r, sem_o):
    valbufs = (valbuf0, valbuf1)
    wid = lax.axis_index("s") * NC + lax.axis_index("c")
    p0 = wid * PAIRS_W
    pending = []

    def wait_slot(slot):
        for i, (s, src, dst, sem) in enumerate(pending):
            if s == slot:
                pltpu.make_async_copy(src, dst, sem).wait()
                pending.pop(i)
                return

    g = 0
    for k in range(PAIRS_W):
        p = p0 + k
        f = lax.div(p, D)
        d = lax.rem(p, D)
        if k == 0:
            pltpu.sync_copy(xt_hbm.at[f, pl.ds(0, B)], idxbuf)
        else:
            fprev = lax.div(p - 1, D)

            @pl.when(f != fprev)
            def _():
                pltpu.sync_copy(xt_hbm.at[f, pl.ds(0, B)], idxbuf)

        pltpu.sync_copy(tt_hbm.at[f, d, :], rowbuf)

        for c in range(NCH):
            slot = g % 2
            wait_slot(slot)
            vslot = valbufs[slot]

            def gbody(jj, _, _c=c, _vs=vslot):
                o = jj * (16 * UNROLL)
                for u in range(UNROLL):
                    oo = o + u * 16
                    v = idxbuf[pl.ds(_c * IB + oo, 16)]
                    _vs[pl.ds(oo, 16)] = plsc.load_gather(rowbuf, [v])
                return 0

            lax.fori_loop(0, IB // (16 * UNROLL), gbody, 0)
            dst = ot_hbm.at[f, d, pl.ds(c * IB, IB)]
            sem = sem_o.at[slot]
            pltpu.async_copy(vslot, dst, sem)
            pending.append((slot, vslot, dst, sem))
            g += 1

    for slot in (0, 1):
        wait_slot(slot)


@jax.jit
def kernel(X, tables):
    xt = X.T
    tt = jnp.transpose(tables, (0, 2, 1))
    mesh = plsc.VectorSubcoreMesh(core_axis_name="c", subcore_axis_name="s")
    ot = pl.kernel(
        _emb_kernel,
        out_type=jax.ShapeDtypeStruct((F, D, B), jnp.float32),
        mesh=mesh,
        scratch_types=[
            pltpu.VMEM((V,), jnp.float32),
            pltpu.VMEM((B,), jnp.int32),
            pltpu.VMEM((IB,), jnp.float32),
            pltpu.VMEM((IB,), jnp.float32),
            pltpu.SemaphoreType.DMA,
            pltpu.SemaphoreType.DMA((2,)),
        ],
        compiler_params=pltpu.CompilerParams(needs_layout_passes=False),
    )(xt, tt)
    return jnp.transpose(ot, (2, 0, 1))


# speedup vs baseline: 1.1680x; 1.0194x over previous
import functools

import jax
import jax.numpy as jnp
from jax import lax
from jax.experimental import pallas as pl
from jax.experimental.pallas import tpu as pltpu
from jax.experimental.pallas import tpu_sc as plsc

B = 16384
F = 26
V = 100000
D = 16

NC = 2
NS = 16
NW = NC * NS

PAIRS = F * D
PAIRS_W = PAIRS // NW
IB = 4096
NCH = B // IB
UNROLL = 8


def _emb_kernel(xt_hbm, tt_hbm, ot_hbm, rowbuf, idxbuf, valbuf0, valbuf1, sem_r, sem_o):
    valbufs = (valbuf0, valbuf1)
    wid = lax.axis_index("s") * NC + lax.axis_index("c")
    p0 = wid * PAIRS_W
    pending = []

    def wait_slot(slot):
        for i, (s, src, dst, sem) in enumerate(pending):
            if s == slot:
                pltpu.make_async_copy(src, dst, sem).wait()
                pending.pop(i)
                return

    g = 0
    for k in range(PAIRS_W):
        p = p0 + k
        f = lax.div(p, D)
        d = lax.rem(p, D)
        if k == 0:
            pltpu.sync_copy(xt_hbm.at[f, pl.ds(0, B)], idxbuf)
        else:
            fprev = lax.div(p - 1, D)

            @pl.when(f != fprev)
            def _():
                pltpu.sync_copy(xt_hbm.at[f, pl.ds(0, B)], idxbuf)

        pltpu.sync_copy(tt_hbm.at[f, d, :], rowbuf)

        for c in range(NCH):
            slot = g % 2
            wait_slot(slot)
            vslot = valbufs[slot]

            def gbody(jj, _, _c=c, _vs=vslot):
                o = jj * (16 * UNROLL)
                for u in range(UNROLL):
                    oo = o + u * 16
                    v = idxbuf[pl.ds(_c * IB + oo, 16)]
                    _vs[pl.ds(oo, 16)] = plsc.load_gather(rowbuf, [v])
                return 0

            lax.fori_loop(0, IB // (16 * UNROLL), gbody, 0)
            dst = ot_hbm.at[f, d, pl.ds(c * IB, IB)]
            sem = sem_o.at[slot]
            pltpu.async_copy(vslot, dst, sem)
            pending.append((slot, vslot, dst, sem))
            g += 1

    for slot in (0, 1):
        wait_slot(slot)


@jax.jit
def kernel(X, tables):
    xt = X.T
    tt = jnp.transpose(tables, (0, 2, 1))
    mesh = plsc.VectorSubcoreMesh(core_axis_name="c", subcore_axis_name="s")
    ot = pl.kernel(
        _emb_kernel,
        out_type=jax.ShapeDtypeStruct((F, D, B), jnp.float32),
        mesh=mesh,
        scratch_types=[
            pltpu.VMEM((V,), jnp.float32),
            pltpu.VMEM((B,), jnp.int32),
            pltpu.VMEM((IB,), jnp.float32),
            pltpu.VMEM((IB,), jnp.float32),
            pltpu.SemaphoreType.DMA,
            pltpu.SemaphoreType.DMA((2,)),
        ],
        compiler_params=pltpu.CompilerParams(needs_layout_passes=False),
    )(xt, tt)
    return jnp.transpose(ot, (2, 0, 1))


# speedup vs baseline: 1.5901x; 1.3613x over previous
import functools

import jax
import jax.numpy as jnp
from jax import lax
from jax.experimental import pallas as pl
from jax.experimental.pallas import tpu as pltpu
from jax.experimental.pallas import tpu_sc as plsc

B = 16384
F = 26
V = 100000
D = 16

NC = 2
NS = 16
NW = NC * NS

PAIRS = F * D
PAIRS_W = PAIRS // NW
IB = 4096
NCH = B // IB
UNROLL = 8


def _emb_kernel(xt_hbm, tt_hbm, ot_hbm, rowbuf, idxbuf, valbuf0, valbuf1, sem_r, sem_o):
    valbufs = (valbuf0, valbuf1)
    wid = lax.axis_index("s") * NC + lax.axis_index("c")
    p0 = wid * PAIRS_W
    pending = []

    def wait_slot(slot):
        for i, (s, src, dst, sem) in enumerate(pending):
            if s == slot:
                pltpu.make_async_copy(src, dst, sem).wait()
                pending.pop(i)
                return

    g = 0
    for k in range(PAIRS_W):
        p = p0 + k
        f = lax.div(p, D)
        d = lax.rem(p, D)
        if k == 0:
            pltpu.sync_copy(xt_hbm.at[f, pl.ds(0, B)], idxbuf)
        else:
            fprev = lax.div(p - 1, D)

            @pl.when(f != fprev)
            def _():
                pltpu.sync_copy(xt_hbm.at[f, pl.ds(0, B)], idxbuf)

        pltpu.sync_copy(tt_hbm.at[f, d, :], rowbuf)

        for c in range(NCH):
            slot = g % 2
            wait_slot(slot)
            vslot = valbufs[slot]

            def gbody(jj, _c=c, _vs=vslot):
                o = jj * 16
                v = idxbuf[pl.ds(_c * IB + o, 16)]
                _vs[pl.ds(o, 16)] = plsc.load_gather(rowbuf, [v])

            plsc.parallel_loop(0, IB // 16, 1, unroll=UNROLL)(gbody)
            dst = ot_hbm.at[f, d, pl.ds(c * IB, IB)]
            sem = sem_o.at[slot]
            pltpu.async_copy(vslot, dst, sem)
            pending.append((slot, vslot, dst, sem))
            g += 1

    for slot in (0, 1):
        wait_slot(slot)


@jax.jit
def kernel(X, tables):
    xt = X.T
    tt = jnp.transpose(tables, (0, 2, 1))
    mesh = plsc.VectorSubcoreMesh(core_axis_name="c", subcore_axis_name="s")
    ot = pl.kernel(
        _emb_kernel,
        out_type=jax.ShapeDtypeStruct((F, D, B), jnp.float32),
        mesh=mesh,
        scratch_types=[
            pltpu.VMEM((V,), jnp.float32),
            pltpu.VMEM((B,), jnp.int32),
            pltpu.VMEM((IB,), jnp.float32),
            pltpu.VMEM((IB,), jnp.float32),
            pltpu.SemaphoreType.DMA,
            pltpu.SemaphoreType.DMA((2,)),
        ],
        compiler_params=pltpu.CompilerParams(needs_layout_passes=False),
    )(xt, tt)
    return jnp.transpose(ot, (2, 0, 1))
